# baseline (speedup 1.0000x reference)
import jax
import jax.numpy as jnp
from jax.experimental import pallas as pl
from jax.experimental.pallas import tpu as pltpu

_D = 128
_N = 100
_B = 1024
_BB = 32
_TEMP = 1.0


_LAG = 4
_Q = (_N + 1 + _LAG - 1) // _LAG


def _body(env_idx_ref, w_ref, deltas_ref, a_ref, wb_ref, ta_ref):
    i = pl.program_id(0)
    w = w_ref[...]

    for q in range(_LAG):
        lo = q * _Q
        hi = min(lo + _Q, _N)

        @pl.when(i == q)
        def _tables():
            row = jax.lax.broadcasted_iota(jnp.int32, (_D, _D), 0)
            col = jax.lax.broadcasted_iota(jnp.int32, (_D, _D), 1)
            mask = jnp.where(row == col, 0.0, 1.0)
            ta_ref[lo:hi] = (
                jax.nn.sigmoid((w[None] + deltas_ref[lo:hi]) * (1.0 / _TEMP)) * mask[None]
            )
            if q == _LAG - 1:
                ta_ref[_N] = jax.nn.sigmoid(w * (1.0 / _TEMP)) * mask

    @pl.when((i > 0) & (i <= _B // _BB))
    def _emit_wb():
        base = (i - 1) * _BB
        for j in range(_BB):
            e = env_idx_ref[base + j]
            idx = jnp.clip(e, 0, _N - 1)
            valid = e < _N
            wb_ref[j] = w + jnp.where(valid, 1.0, 0.0) * deltas_ref[idx]

    @pl.when(i > _LAG)
    def _emit_a():
        base = (i - 1 - _LAG) * _BB
        for j in range(_BB):
            e = env_idx_ref[base + j]
            idx = jnp.clip(e, 0, _N - 1)
            valid = e < _N
            a_ref[j] = ta_ref[jnp.where(valid, idx, _N)]


@jax.jit
def _run(env_idx, W_adj, env_deltas):
    grid = (1 + _LAG + _B // _BB,)
    out_shape = (
        jax.ShapeDtypeStruct((_B, _D, _D), jnp.float32),
        jax.ShapeDtypeStruct((_B, _D, _D), jnp.float32),
    )
    a_map = lambda i: (jnp.clip(i - 1 - _LAG, 0, _B // _BB - 1), 0, 0)
    wb_map = lambda i: (jnp.clip(i - 1, 0, _B // _BB - 1), 0, 0)
    return pl.pallas_call(
        _body,
        grid=grid,
        in_specs=[
            pl.BlockSpec(memory_space=pltpu.SMEM),
            pl.BlockSpec((_D, _D), lambda i: (0, 0)),
            pl.BlockSpec((_N, _D, _D), lambda i: (0, 0, 0)),
        ],
        out_specs=[
            pl.BlockSpec((_BB, _D, _D), a_map),
            pl.BlockSpec((_BB, _D, _D), wb_map),
        ],
        out_shape=out_shape,
        scratch_shapes=[pltpu.VMEM((_N + 1, _D, _D), jnp.float32)],
    )(env_idx, W_adj, env_deltas)


def kernel(env_idx, W_adj, env_deltas):
    return _run(env_idx, W_adj, env_deltas)
